# baseline (speedup 1.0000x reference)
import jax
import jax.numpy as jnp
from jax import lax
from jax.experimental import pallas as pl
from jax.experimental.pallas import tpu as pltpu
from jax.experimental.pallas import tpu_sc as plsc

GRID_N = 27
PATCHES = GRID_N * GRID_N
D = 768
BATCH = 128

NC = 2
NS = 16
LANES = 16
NW = NC * NS

CP = 24
LAST_P0 = 728 - CP
TAIL_P0 = 728
NBUF = 4
VREGS_PER_ROW = D // LANES


def _pos_body(row_ref, col_ref, pos_ref):
    row = row_ref[...]
    col = col_ref[...]
    rr = jnp.reshape(
        jax.lax.broadcast_in_dim(row, (GRID_N, GRID_N, D), (0, 2)),
        (PATCHES, D),
    )
    cc = jnp.reshape(
        jax.lax.broadcast_in_dim(col, (GRID_N, GRID_N, D), (1, 2)),
        (PATCHES, D),
    )
    pos_ref[...] = rr + cc


def _sc_body(x_hbm, pos_hbm, out_hbm, pos_v, bufs,
             s_in0, s_in1, s_in2, s_in3, s_out0, s_out1, s_out2, s_out3):
    sem_in = [s_in0, s_in1, s_in2, s_in3]
    sem_out = [s_out0, s_out1, s_out2, s_out3]

    wid = lax.axis_index("s") * NC + lax.axis_index("c")
    is_tail = wid == NW - 1
    p0 = pl.multiple_of(
        jnp.where(is_tail, TAIL_P0, jnp.minimum(wid * CP, LAST_P0)), 8)

    def run(cp):
        pltpu.async_copy(
            pos_hbm.at[pl.ds(p0, cp)], pos_v.at[pl.ds(0, cp)], sem_in[0]
        ).wait()

        def in_copy(b, k):
            return pltpu.make_async_copy(
                x_hbm.at[b, pl.ds(p0, cp)],
                bufs.at[k, pl.ds(0, cp)], sem_in[k])

        def out_copy(b, k):
            return pltpu.make_async_copy(
                bufs.at[k, pl.ds(0, cp)],
                out_hbm.at[b, pl.ds(p0, cp)], sem_out[k])

        def add_pos(k):
            def add_vec(t, _):
                r = t // VREGS_PER_ROW
                j = t - r * VREGS_PER_ROW
                sl = pl.ds(j * LANES, LANES)
                plsc.addupdate(bufs.at[k, r, sl], pos_v[r, sl])
                return 0

            lax.fori_loop(0, cp * VREGS_PER_ROW, add_vec, 0)

        def round_body(i, _):
            b0 = i * NBUF
            for k in range(NBUF):
                @pl.when(i > 0)
                def _():
                    out_copy(b0 - NBUF + k, k).wait()

                in_copy(b0 + k, k).start()
            for k in range(NBUF):
                in_copy(b0 + k, k).wait()
                add_pos(k)
                out_copy(b0 + k, k).start()
            return 0

        lax.fori_loop(0, BATCH // NBUF, round_body, 0)
        for k in range(NBUF):
            out_copy(BATCH - NBUF + k, k).wait()

    @pl.when(jnp.logical_not(is_tail))
    def _():
        run(CP)

    @pl.when(is_tail)
    def _():
        run(1)


def kernel(x, row_emb, col_emb):
    pos = pl.pallas_call(
        _pos_body,
        out_shape=jax.ShapeDtypeStruct((PATCHES, D), x.dtype),
    )(row_emb, col_emb)

    mesh = plsc.VectorSubcoreMesh(core_axis_name="c", subcore_axis_name="s")
    f = pl.kernel(
        _sc_body,
        out_type=jax.ShapeDtypeStruct(x.shape, x.dtype),
        mesh=mesh,
        scratch_types=[
            pltpu.VMEM((CP, D), jnp.float32),
            pltpu.VMEM((NBUF, CP, D), jnp.float32),
        ] + [pltpu.SemaphoreType.DMA] * (2 * NBUF),
    )
    return f(x, pos)


# speedup vs baseline: 1.5146x; 1.5146x over previous
import jax
import jax.numpy as jnp
from jax import lax
from jax.experimental import pallas as pl
from jax.experimental.pallas import tpu as pltpu
from jax.experimental.pallas import tpu_sc as plsc

GRID_N = 27
PATCHES = GRID_N * GRID_N
D = 768
BATCH = 128

NC = 2
NS = 16
LANES = 16
NW = NC * NS

CP = 24
LAST_P0 = 728 - CP
TAIL_P0 = 728
NBUF = 4
VREGS_PER_ROW = D // LANES


def _pos_body(row_ref, col_ref, pos_ref):
    row = row_ref[...]
    col = col_ref[...]
    rr = jnp.reshape(
        jax.lax.broadcast_in_dim(row, (GRID_N, GRID_N, D), (0, 2)),
        (PATCHES, D),
    )
    cc = jnp.reshape(
        jax.lax.broadcast_in_dim(col, (GRID_N, GRID_N, D), (1, 2)),
        (PATCHES, D),
    )
    pos_ref[...] = rr + cc


def _sc_body(x_hbm, pos_hbm, out_hbm, pos_v, bufs,
             s_in0, s_in1, s_in2, s_in3, s_out0, s_out1, s_out2, s_out3):
    sem_in = [s_in0, s_in1, s_in2, s_in3]
    sem_out = [s_out0, s_out1, s_out2, s_out3]

    wid = lax.axis_index("s") * NC + lax.axis_index("c")
    is_tail = wid == NW - 1
    p0 = pl.multiple_of(
        jnp.where(is_tail, TAIL_P0, jnp.minimum(wid * CP, LAST_P0)), 8)

    def run(cp):
        pltpu.async_copy(
            pos_hbm.at[pl.ds(p0, cp)], pos_v.at[pl.ds(0, cp)], sem_in[0]
        ).wait()

        def in_copy(b, k):
            return pltpu.make_async_copy(
                x_hbm.at[b, pl.ds(p0, cp)],
                bufs.at[k, pl.ds(0, cp)], sem_in[k])

        def out_copy(b, k):
            return pltpu.make_async_copy(
                bufs.at[k, pl.ds(0, cp)],
                out_hbm.at[b, pl.ds(p0, cp)], sem_out[k])

        def add_pos(k):
            def add_row(r, _):
                for j in range(VREGS_PER_ROW):
                    sl = pl.ds(j * LANES, LANES)
                    plsc.addupdate(bufs.at[k, r, sl], pos_v[r, sl])
                return 0

            lax.fori_loop(0, cp, add_row, 0)

        def round_body(i, _):
            b0 = i * NBUF
            for k in range(NBUF):
                @pl.when(i > 0)
                def _():
                    out_copy(b0 - NBUF + k, k).wait()

                in_copy(b0 + k, k).start()
            for k in range(NBUF):
                in_copy(b0 + k, k).wait()
                add_pos(k)
                out_copy(b0 + k, k).start()
            return 0

        lax.fori_loop(0, BATCH // NBUF, round_body, 0)
        for k in range(NBUF):
            out_copy(BATCH - NBUF + k, k).wait()

    @pl.when(jnp.logical_not(is_tail))
    def _():
        run(CP)

    @pl.when(is_tail)
    def _():
        run(1)


def kernel(x, row_emb, col_emb):
    pos = pl.pallas_call(
        _pos_body,
        out_shape=jax.ShapeDtypeStruct((PATCHES, D), x.dtype),
    )(row_emb, col_emb)

    mesh = plsc.VectorSubcoreMesh(core_axis_name="c", subcore_axis_name="s")
    f = pl.kernel(
        _sc_body,
        out_type=jax.ShapeDtypeStruct(x.shape, x.dtype),
        mesh=mesh,
        scratch_types=[
            pltpu.VMEM((CP, D), jnp.float32),
            pltpu.VMEM((NBUF, CP, D), jnp.float32),
        ] + [pltpu.SemaphoreType.DMA] * (2 * NBUF),
    )
    return f(x, pos)


# speedup vs baseline: 1.5234x; 1.0058x over previous
import jax
import jax.numpy as jnp
from jax import lax
from jax.experimental import pallas as pl
from jax.experimental.pallas import tpu as pltpu
from jax.experimental.pallas import tpu_sc as plsc

GRID_N = 27
PATCHES = GRID_N * GRID_N
D = 768
BATCH = 128

NC = 2
NS = 16
LANES = 16
NW = NC * NS

CP = 24
LAST_P0 = 728 - CP
TAIL_P0 = 728
NBUF = 2
BPC = 2
NCHUNK = BATCH // BPC
VREGS_PER_ROW = D // LANES


def _pos_body(row_ref, col_ref, pos_ref):
    row = row_ref[...]
    col = col_ref[...]
    rr = jnp.reshape(
        jax.lax.broadcast_in_dim(row, (GRID_N, GRID_N, D), (0, 2)),
        (PATCHES, D),
    )
    cc = jnp.reshape(
        jax.lax.broadcast_in_dim(col, (GRID_N, GRID_N, D), (1, 2)),
        (PATCHES, D),
    )
    pos_ref[...] = rr + cc


def _sc_body(x_hbm, pos_hbm, out_hbm, pos_v, bufs, *sems):
    sem_in = list(sems[:NBUF])
    sem_out = list(sems[NBUF:])

    wid = lax.axis_index("s") * NC + lax.axis_index("c")
    is_tail = wid == NW - 1
    p0 = pl.multiple_of(
        jnp.where(is_tail, TAIL_P0, jnp.minimum(wid * CP, LAST_P0)), 8)

    def run(cp):
        pltpu.async_copy(
            pos_hbm.at[pl.ds(p0, cp)], pos_v.at[pl.ds(0, cp)], sem_in[0]
        ).wait()

        def in_copy(c, k):
            return pltpu.make_async_copy(
                x_hbm.at[pl.ds(c * BPC, BPC), pl.ds(p0, cp)],
                bufs.at[k, :, pl.ds(0, cp)], sem_in[k])

        def out_copy(c, k):
            return pltpu.make_async_copy(
                bufs.at[k, :, pl.ds(0, cp)],
                out_hbm.at[pl.ds(c * BPC, BPC), pl.ds(p0, cp)], sem_out[k])

        def add_pos(k):
            def add_row(r, _):
                for b1 in range(BPC):
                    for j in range(VREGS_PER_ROW):
                        sl = pl.ds(j * LANES, LANES)
                        plsc.addupdate(bufs.at[k, b1, r, sl], pos_v[r, sl])
                return 0

            lax.fori_loop(0, cp, add_row, 0)

        def round_body(i, _):
            c0 = i * NBUF
            for k in range(NBUF):
                @pl.when(i > 0)
                def _():
                    out_copy(c0 - NBUF + k, k).wait()

                in_copy(c0 + k, k).start()
            for k in range(NBUF):
                in_copy(c0 + k, k).wait()
                add_pos(k)
                out_copy(c0 + k, k).start()
            return 0

        lax.fori_loop(0, NCHUNK // NBUF, round_body, 0)
        for k in range(NBUF):
            out_copy(NCHUNK - NBUF + k, k).wait()

    @pl.when(jnp.logical_not(is_tail))
    def _():
        run(CP)

    @pl.when(is_tail)
    def _():
        run(1)


def kernel(x, row_emb, col_emb):
    pos = pl.pallas_call(
        _pos_body,
        out_shape=jax.ShapeDtypeStruct((PATCHES, D), x.dtype),
    )(row_emb, col_emb)

    mesh = plsc.VectorSubcoreMesh(core_axis_name="c", subcore_axis_name="s")
    f = pl.kernel(
        _sc_body,
        out_type=jax.ShapeDtypeStruct(x.shape, x.dtype),
        mesh=mesh,
        scratch_types=[
            pltpu.VMEM((CP, D), jnp.float32),
            pltpu.VMEM((NBUF, BPC, CP, D), jnp.float32),
        ] + [pltpu.SemaphoreType.DMA] * (2 * NBUF),
    )
    return f(x, pos)


# speedup vs baseline: 1.6595x; 1.0894x over previous
import jax
import jax.numpy as jnp
from jax import lax
from jax.experimental import pallas as pl
from jax.experimental.pallas import tpu as pltpu
from jax.experimental.pallas import tpu_sc as plsc

GRID_N = 27
PATCHES = GRID_N * GRID_N
D = 768
BATCH = 128

NC = 2
NS = 16
LANES = 16
NW = NC * NS

CP = 24
LAST_P0 = 728 - CP
TAIL_P0 = 728
NBUF = 2
BPC = 2
NCHUNK = BATCH // BPC
VREGS_PER_ROW = D // LANES


def _pos_body(row_ref, col_ref, pos_ref):
    row = row_ref[...]
    col = col_ref[...]
    rr = jnp.reshape(
        jax.lax.broadcast_in_dim(row, (GRID_N, GRID_N, D), (0, 2)),
        (PATCHES, D),
    )
    cc = jnp.reshape(
        jax.lax.broadcast_in_dim(col, (GRID_N, GRID_N, D), (1, 2)),
        (PATCHES, D),
    )
    pos_ref[...] = rr + cc


def _sc_body(x_hbm, pos_hbm, out_hbm, pos_v, bufs, *sems):
    sem_in = list(sems[:NBUF])
    sem_out = list(sems[NBUF:])

    wid = lax.axis_index("s") * NC + lax.axis_index("c")
    is_tail = wid == NW - 1
    p0 = pl.multiple_of(
        jnp.where(is_tail, TAIL_P0, jnp.minimum(wid * CP, LAST_P0)), 8)

    def run(cp):
        pltpu.async_copy(
            pos_hbm.at[pl.ds(p0, cp)], pos_v.at[pl.ds(0, cp)], sem_in[0]
        ).wait()

        def in_copy(c, k):
            return pltpu.make_async_copy(
                x_hbm.at[pl.ds(c * BPC, BPC), pl.ds(p0, cp)],
                bufs.at[k, :, pl.ds(0, cp)], sem_in[k])

        def out_copy(c, k):
            return pltpu.make_async_copy(
                bufs.at[k, :, pl.ds(0, cp)],
                out_hbm.at[pl.ds(c * BPC, BPC), pl.ds(p0, cp)], sem_out[k])

        def add_pos(k):
            GRP = 8

            def add_row(r, _):
                for b1 in range(BPC):
                    for g in range(0, VREGS_PER_ROW, GRP):
                        sls = [pl.ds((g + j) * LANES, LANES)
                               for j in range(GRP)]
                        vals = [pos_v[r, sl] for sl in sls]
                        for sl, v in zip(sls, vals):
                            plsc.addupdate(bufs.at[k, b1, r, sl], v)
                return 0

            lax.fori_loop(0, cp, add_row, 0)

        def round_body(i, _):
            c0 = i * NBUF
            for k in range(NBUF):
                @pl.when(i > 0)
                def _():
                    out_copy(c0 - NBUF + k, k).wait()

                in_copy(c0 + k, k).start()
            for k in range(NBUF):
                in_copy(c0 + k, k).wait()
                add_pos(k)
                out_copy(c0 + k, k).start()
            return 0

        lax.fori_loop(0, NCHUNK // NBUF, round_body, 0)
        for k in range(NBUF):
            out_copy(NCHUNK - NBUF + k, k).wait()

    @pl.when(jnp.logical_not(is_tail))
    def _():
        run(CP)

    @pl.when(is_tail)
    def _():
        run(1)


def kernel(x, row_emb, col_emb):
    pos = pl.pallas_call(
        _pos_body,
        out_shape=jax.ShapeDtypeStruct((PATCHES, D), x.dtype),
    )(row_emb, col_emb)

    mesh = plsc.VectorSubcoreMesh(core_axis_name="c", subcore_axis_name="s")
    f = pl.kernel(
        _sc_body,
        out_type=jax.ShapeDtypeStruct(x.shape, x.dtype),
        mesh=mesh,
        scratch_types=[
            pltpu.VMEM((CP, D), jnp.float32),
            pltpu.VMEM((NBUF, BPC, CP, D), jnp.float32),
        ] + [pltpu.SemaphoreType.DMA] * (2 * NBUF),
    )
    return f(x, pos)
